# baseline (speedup 1.0000x reference)
import dataclasses
import functools

import jax
import jax.numpy as jnp
from jax import lax
from jax.experimental import pallas as pl
from jax.experimental.pallas import tpu as pltpu
from jax.experimental.pallas import tpu_sc as plsc

EMB = 64
SEQ = 200
BATCH = 4096
NUM_CORES = 2
NUM_SUBCORES = 16
NW = NUM_CORES * NUM_SUBCORES
LANES = 16
BTILE = 128


def _compiler_params():
    cp = pltpu.CompilerParams(use_tc_tiling_on_sc=False)
    if "needs_layout_passes" in pltpu.CompilerParams.__dataclass_fields__:
        cp = dataclasses.replace(cp, needs_layout_passes=False)
    return cp


def _emb_kernel():
    mesh = plsc.VectorSubcoreMesh(core_axis_name="c", subcore_axis_name="s")
    out_shape = (SEQ, EMB // 8, BATCH // BTILE, 8 * BTILE)

    @functools.partial(
        pl.kernel,
        out_type=jax.ShapeDtypeStruct(out_shape, jnp.float32),
        mesh=mesh,
        compiler_params=_compiler_params(),
        scratch_types=[
            pltpu.VMEM((SEQ, BTILE), jnp.int32),
            pltpu.VMEM((BTILE, EMB), jnp.float32),
            pltpu.VMEM((BTILE, EMB), jnp.float32),
            pltpu.VMEM((EMB // 8, 8 * BTILE), jnp.float32),
            pltpu.VMEM((EMB // 8, 8 * BTILE), jnp.float32),
            pltpu.VMEM((SEQ, EMB), jnp.float32),
            pltpu.SemaphoreType.DMA,
            pltpu.SemaphoreType.DMA,
            pltpu.SemaphoreType.DMA,
            pltpu.SemaphoreType.DMA,
        ],
    )
    def k(xr_hbm, tok_hbm, pos_hbm, out_hbm,
          idx_v, rows0, rows1, tr0, tr1, pos_v, g0, g1, o0, o1):
        wid = lax.axis_index("s") * NUM_CORES + lax.axis_index("c")
        pltpu.sync_copy(pos_hbm, pos_v)
        pltpu.sync_copy(xr_hbm.at[wid], idx_v)

        lane = lax.iota(jnp.int32, LANES)
        rbs = [lane + jnp.int32(rb * LANES) for rb in range(BTILE // LANES)]

        def transpose_add(s, rows_v, tr_v):
            s_vec = jnp.full((LANES,), 0, jnp.int32) + s
            for j in range(EMB // LANES):
                @plsc.parallel_loop(0, LANES, unroll=2)
                def _k(kk):
                    e = jnp.int32(16 * j) + lax.bitwise_and(lane + kk, jnp.int32(15))
                    e_hi = lax.shift_right_logical(e, jnp.int32(3))
                    e_lo7 = lax.shift_left(lax.bitwise_and(e, jnp.int32(7)),
                                           jnp.int32(7))
                    pp = plsc.load_gather(pos_v, [s_vec, e])
                    for rb in range(BTILE // LANES):
                        v = plsc.load_gather(rows_v, [rbs[rb], e]) + pp
                        plsc.store_scatter(tr_v, [e_hi, e_lo7 + rbs[rb]], v)

        def fire_gather(s, rows_v, sem):
            pltpu.async_copy(tok_hbm.at[idx_v.at[s]], rows_v, sem)

        def wait_gather(s, rows_v, sem):
            pltpu.make_async_copy(tok_hbm.at[idx_v.at[s]], rows_v, sem).wait()

        def fire_out(s, tr_v, sem):
            pltpu.async_copy(tr_v, out_hbm.at[s, pl.ds(0, EMB // 8), wid], sem)

        def wait_out(tr_v, sem):
            pltpu.make_async_copy(
                tr_v, out_hbm.at[0, pl.ds(0, EMB // 8), 0], sem).wait()

        fire_gather(0, rows0, g0)

        @pl.loop(0, SEQ // 2)
        def _pair(p):
            s_a = 2 * p

            @pl.when(p > 0)
            def _():
                wait_out(tr1, o1)
            fire_gather(s_a + 1, rows1, g1)
            wait_gather(s_a, rows0, g0)
            transpose_add(s_a, rows0, tr0)
            fire_out(s_a, tr0, o0)

            @pl.when(p < SEQ // 2 - 1)
            def _():
                wait_out(tr0, o0)
                fire_gather(s_a + 2, rows0, g0)
            wait_gather(s_a + 1, rows1, g1)
            transpose_add(s_a + 1, rows1, tr1)
            fire_out(s_a + 1, tr1, o1)

        wait_out(tr0, o0)
        wait_out(tr1, o1)

    return k


def kernel(x, token_table, pos_table):
    b, seq = x.shape
    xr = x.reshape(b // BTILE, BTILE, seq).transpose(0, 2, 1)
    out = _emb_kernel()(xr, token_table, pos_table)
    out = out.reshape(seq, EMB // 8, b // BTILE, 8, BTILE)
    out = out.transpose(2, 4, 0, 1, 3).reshape(b, seq, EMB)
    return out
